# baseline (speedup 1.0000x reference)
import functools

import jax
import jax.numpy as jnp
from jax import lax
from jax.experimental import pallas as pl
from jax.experimental.pallas import tpu as pltpu

N = 3200000
IN_CH = 10
OUT_CH = 32
NUM_SEGMENTS = 100000
EPS = 1e-3
NUM_TILES = 32
SEGS_PER_TILE = NUM_SEGMENTS // NUM_TILES

STATS_BLOCK = 6400


def _stats_body(x_ref, ids_ref, wt_ref, s1_ref, s2_ref, cnt_ref):
    i = pl.program_id(0)

    @pl.when(i == 0)
    def _init():
        s1_ref[...] = jnp.zeros_like(s1_ref)
        s2_ref[...] = jnp.zeros_like(s2_ref)
        cnt_ref[...] = jnp.zeros_like(cnt_ref)

    v = jnp.dot(x_ref[...], wt_ref[...], preferred_element_type=jnp.float32)
    s1_ref[...] += jnp.sum(v, axis=0, keepdims=True)
    s2_ref[...] += jnp.sum(v * v, axis=0, keepdims=True)

    ids = ids_ref[0]
    thr = (lax.broadcasted_iota(jnp.int32, (NUM_TILES, 1), 0) + 1) * SEGS_PER_TILE
    below = (ids < thr).astype(jnp.int32)
    cnt_ref[...] += jnp.sum(below, axis=1, keepdims=True).reshape(1, NUM_TILES)


def _stats_pass(inputs, ids3, wt):
    nb = N // STATS_BLOCK
    return pl.pallas_call(
        _stats_body,
        grid=(nb,),
        in_specs=[
            pl.BlockSpec((STATS_BLOCK, IN_CH), lambda i: (i, 0)),
            pl.BlockSpec((1, 1, STATS_BLOCK), lambda i: (i, 0, 0)),
            pl.BlockSpec((IN_CH, OUT_CH), lambda i: (0, 0)),
        ],
        out_specs=[
            pl.BlockSpec((1, OUT_CH), lambda i: (0, 0)),
            pl.BlockSpec((1, OUT_CH), lambda i: (0, 0)),
            pl.BlockSpec((1, NUM_TILES), lambda i: (0, 0)),
        ],
        out_shape=[
            jax.ShapeDtypeStruct((1, OUT_CH), jnp.float32),
            jax.ShapeDtypeStruct((1, OUT_CH), jnp.float32),
            jax.ShapeDtypeStruct((1, NUM_TILES), jnp.int32),
        ],
    )(inputs, ids3, wt)


X_BLOCK = 6400


def _x_body(x_ref, wt_ref, b_ref, o_ref):
    v = jnp.dot(x_ref[...], wt_ref[...], preferred_element_type=jnp.float32)
    o_ref[...] = jnp.maximum(v + b_ref[...], 0.0)


def _x_pass(inputs, wt2, b2):
    nb = N // X_BLOCK
    return pl.pallas_call(
        _x_body,
        grid=(nb,),
        in_specs=[
            pl.BlockSpec((X_BLOCK, IN_CH), lambda i: (i, 0)),
            pl.BlockSpec((IN_CH, OUT_CH), lambda i: (0, 0)),
            pl.BlockSpec((1, OUT_CH), lambda i: (0, 0)),
        ],
        out_specs=pl.BlockSpec((X_BLOCK, OUT_CH), lambda i: (i, 0)),
        out_shape=jax.ShapeDtypeStruct((N, OUT_CH), jnp.float32),
    )(inputs, wt2, b2)


def kernel(inputs, unq_inv, W, b, gamma, beta):
    ids = unq_inv.astype(jnp.int32)
    ids3 = ids.reshape(N // STATS_BLOCK, 1, STATS_BLOCK)
    wt = W.T

    s1, s2, cnt = _stats_pass(inputs, ids3, wt)
    s1 = s1[0]
    s2 = s2[0]

    mean = s1 / N + b
    var = s2 / N - (s1 / N) ** 2
    a = gamma * lax.rsqrt(var + EPS)
    d = beta - mean * a
    wt2 = wt * a[None, :]
    b2 = (b * a + d).reshape(1, OUT_CH)

    x = _x_pass(inputs, wt2, b2)

    x_max = jax.ops.segment_max(x, ids, num_segments=NUM_SEGMENTS)
    out = (x + x_max[ids]) / 2.0
    return out


# speedup vs baseline: 2.4325x; 2.4325x over previous
import functools

import jax
import jax.numpy as jnp
from jax import lax
from jax.experimental import pallas as pl
from jax.experimental.pallas import tpu as pltpu
from jax.experimental.pallas import tpu_sc as plsc

N = 3200000
IN_CH = 10
OUT_CH = 32
NUM_SEGMENTS = 100000
EPS = 1e-3
NUM_CORES = 2
NUM_SUBCORES = 16
NUM_TILES = NUM_CORES * NUM_SUBCORES
SEGS_PER_TILE = NUM_SEGMENTS // NUM_TILES

STATS_BLOCK = 6400
X_BLOCK = 6400
C = 256
IDS_SZ = C + 16
HALF = 16


def _stats_body(x_ref, ids_ref, wt_ref, s1_ref, s2_ref, cnt_ref):
    i = pl.program_id(0)

    @pl.when(i == 0)
    def _init():
        s1_ref[...] = jnp.zeros_like(s1_ref)
        s2_ref[...] = jnp.zeros_like(s2_ref)
        cnt_ref[...] = jnp.zeros_like(cnt_ref)

    v = jnp.dot(x_ref[...], wt_ref[...], preferred_element_type=jnp.float32)
    s1_ref[...] += jnp.sum(v, axis=0, keepdims=True)
    s2_ref[...] += jnp.sum(v * v, axis=0, keepdims=True)

    ids = ids_ref[0]
    thr = (lax.broadcasted_iota(jnp.int32, (NUM_TILES, 1), 0) + 1) * SEGS_PER_TILE
    below = (ids < thr).astype(jnp.int32)
    cnt_ref[...] += jnp.sum(below, axis=1, keepdims=True).reshape(1, NUM_TILES)


def _stats_pass(inputs, ids3, wt):
    nb = N // STATS_BLOCK
    return pl.pallas_call(
        _stats_body,
        grid=(nb,),
        in_specs=[
            pl.BlockSpec((STATS_BLOCK, IN_CH), lambda i: (i, 0)),
            pl.BlockSpec((1, 1, STATS_BLOCK), lambda i: (i, 0, 0)),
            pl.BlockSpec((IN_CH, OUT_CH), lambda i: (0, 0)),
        ],
        out_specs=[
            pl.BlockSpec((1, OUT_CH), lambda i: (0, 0)),
            pl.BlockSpec((1, OUT_CH), lambda i: (0, 0)),
            pl.BlockSpec((1, NUM_TILES), lambda i: (0, 0)),
        ],
        out_shape=[
            jax.ShapeDtypeStruct((1, OUT_CH), jnp.float32),
            jax.ShapeDtypeStruct((1, OUT_CH), jnp.float32),
            jax.ShapeDtypeStruct((1, NUM_TILES), jnp.int32),
        ],
    )(inputs, ids3, wt)


def _x_body(x_ref, wt_ref, b_ref, o_ref):
    v = jnp.dot(x_ref[...], wt_ref[...], preferred_element_type=jnp.float32)
    o_ref[...] = jnp.maximum(v + b_ref[...], 0.0)


def _x_pass(inputs, wt2, b2):
    nb = N // X_BLOCK
    return pl.pallas_call(
        _x_body,
        grid=(nb,),
        in_specs=[
            pl.BlockSpec((X_BLOCK, IN_CH), lambda i: (i, 0)),
            pl.BlockSpec((IN_CH, OUT_CH), lambda i: (0, 0)),
            pl.BlockSpec((1, OUT_CH), lambda i: (0, 0)),
        ],
        out_specs=pl.BlockSpec((X_BLOCK, OUT_CH), lambda i: (i, 0)),
        out_shape=jax.ShapeDtypeStruct((N, OUT_CH), jnp.float32),
    )(inputs, wt2, b2)


def _seg_body(x_hbm, ids_hbm, cnt_hbm, out_hbm, table, xbuf, idbuf, obuf,
              cntbuf):
    wid = lax.axis_index("s") * NUM_CORES + lax.axis_index("c")
    base = wid * SEGS_PER_TILE

    pltpu.sync_copy(cnt_hbm, cntbuf.at[0:NUM_TILES])
    r0 = jnp.where(
        wid == 0, 0, cntbuf[pl.ds(jnp.maximum(wid - 1, 0), HALF)][0])
    r1 = cntbuf[pl.ds(wid, HALF)][0]
    nrows = r1 - r0
    nch = (nrows + C - 1) // C

    def stage(lo):
        s8 = pl.multiple_of(
            jnp.minimum((lo // 8) * 8, N - (C + 8)), 8)
        sa = pl.multiple_of(
            jnp.minimum((lo // 16) * 16, N - IDS_SZ), 16)
        x0 = pl.multiple_of(s8 * OUT_CH, 8 * OUT_CH)
        pltpu.sync_copy(x_hbm.at[pl.ds(x0, (C + 8) * OUT_CH)],
                        xbuf.at[0:(C + 8) * OUT_CH])
        pltpu.sync_copy(ids_hbm.at[pl.ds(sa, IDS_SZ)], idbuf.at[0:IDS_SZ])
        return s8, sa

    neg = jnp.full((HALF,), -jnp.inf, dtype=jnp.float32)

    def a_chunk(k, carry):
        lo = r0 + k * C
        hi = jnp.minimum(lo + C, r1)
        s8, sa = stage(lo)

        def a_row(g, carry):
            cur, m0, m1 = carry
            sid = idbuf[pl.ds(g - sa, HALF)][0]
            new = sid != cur
            xo = (g - s8) * OUT_CH
            x0 = xbuf[pl.ds(xo, HALF)]
            x1 = xbuf[pl.ds(xo + HALF, HALF)]
            m0 = jnp.where(new, x0, jnp.maximum(m0, x0))
            m1 = jnp.where(new, x1, jnp.maximum(m1, x1))
            t = (sid - base) * OUT_CH
            table[pl.ds(t, HALF)] = m0
            table[pl.ds(t + HALF, HALF)] = m1
            return (sid, m0, m1)

        return lax.fori_loop(lo, hi, a_row, carry)

    lax.fori_loop(0, nch, a_chunk, (jnp.int32(-1), neg, neg))


    def b_row_compute(g, lo, s8, sa):
        sid = idbuf[pl.ds(g - sa, HALF)][0]
        t = (sid - base) * OUT_CH
        t0 = table[pl.ds(t, HALF)]
        t1 = table[pl.ds(t + HALF, HALF)]
        xo = (g - s8) * OUT_CH
        o = pl.multiple_of((g - lo) * OUT_CH, OUT_CH)
        obuf[pl.ds(o, HALF)] = (xbuf[pl.ds(xo, HALF)] + t0) * 0.5
        obuf[pl.ds(o + HALF, HALF)] = (xbuf[pl.ds(xo + HALF, HALF)] + t1) * 0.5

    nfull = nrows // C
    rem = nrows - nfull * C
    nb_chunks = jnp.where(rem > 0, nfull + 1, nfull)

    def b_chunk(k, _):
        lo = jnp.minimum(r0 + k * C, r1 - C)
        s8, sa = stage(lo)
        lax.fori_loop(
            lo, lo + C, lambda g, c: (b_row_compute(g, lo, s8, sa), 0)[1], 0)
        o0 = pl.multiple_of(lo * OUT_CH, OUT_CH)
        pltpu.sync_copy(obuf, out_hbm.at[pl.ds(o0, C * OUT_CH)])
        return 0

    @pl.when(nrows >= C)
    def _b_main():
        lax.fori_loop(0, nb_chunks, b_chunk, 0)

    @pl.when((nrows > 0) & (nrows < C))
    def _b_small():
        s8, sa = stage(r0)

        def row(g, c):
            b_row_compute(g, r0, s8, sa)
            src = pl.multiple_of((g - r0) * OUT_CH, OUT_CH)
            dst = pl.multiple_of(g * OUT_CH, OUT_CH)
            pltpu.sync_copy(obuf.at[pl.ds(src, OUT_CH)],
                            out_hbm.at[pl.ds(dst, OUT_CH)])
            return c

        lax.fori_loop(r0, r1, row, 0)


@functools.partial(
    pl.kernel,
    out_type=jax.ShapeDtypeStruct((N * OUT_CH,), jnp.float32),
    mesh=plsc.VectorSubcoreMesh(
        core_axis_name="c", subcore_axis_name="s", num_cores=NUM_CORES,
        num_subcores=NUM_SUBCORES),
    scratch_types=[
        pltpu.VMEM((SEGS_PER_TILE * OUT_CH,), jnp.float32),
        pltpu.VMEM(((C + 8) * OUT_CH,), jnp.float32),
        pltpu.VMEM((IDS_SZ + HALF,), jnp.int32),
        pltpu.VMEM((C * OUT_CH,), jnp.float32),
        pltpu.VMEM((NUM_TILES + HALF,), jnp.int32),
    ],
)
def _seg_kernel(x_hbm, ids_hbm, cnt_hbm, out_hbm, table, xbuf, idbuf, obuf,
                cntbuf):
    _seg_body(x_hbm, ids_hbm, cnt_hbm, out_hbm, table, xbuf, idbuf, obuf,
              cntbuf)


def kernel(inputs, unq_inv, W, b, gamma, beta):
    ids = unq_inv.astype(jnp.int32)
    ids3 = ids.reshape(N // STATS_BLOCK, 1, STATS_BLOCK)
    wt = W.T

    s1, s2, cnt = _stats_pass(inputs, ids3, wt)
    s1 = s1[0]
    s2 = s2[0]

    mean = s1 / N + b
    var = s2 / N - (s1 / N) ** 2
    a = gamma * lax.rsqrt(var + EPS)
    d = beta - mean * a
    wt2 = wt * a[None, :]
    b2 = (b * a + d).reshape(1, OUT_CH)

    x = _x_pass(inputs, wt2, b2)

    out_flat = _seg_kernel(x.reshape(N * OUT_CH), ids, cnt[0])
    return out_flat.reshape(N, OUT_CH)


# speedup vs baseline: 4.0044x; 1.6462x over previous
import functools

import jax
import jax.numpy as jnp
from jax import lax
from jax.experimental import pallas as pl
from jax.experimental.pallas import tpu as pltpu
from jax.experimental.pallas import tpu_sc as plsc

N = 3200000
IN_CH = 10
OUT_CH = 32
NUM_SEGMENTS = 100000
EPS = 1e-3
NUM_CORES = 2
NUM_SUBCORES = 16
NUM_TILES = NUM_CORES * NUM_SUBCORES
SEGS_PER_TILE = NUM_SEGMENTS // NUM_TILES

STATS_BLOCK = 25600
X_BLOCK = 25600
C = 128
IDS_SZ = C + 16
HALF = 16
XW = (C + 8) * OUT_CH


def _stats_body(x_ref, ids_ref, wt_ref, s1_ref, s2_ref, cnt_ref):
    i = pl.program_id(0)

    @pl.when(i == 0)
    def _init():
        s1_ref[...] = jnp.zeros_like(s1_ref)
        s2_ref[...] = jnp.zeros_like(s2_ref)
        cnt_ref[...] = jnp.zeros_like(cnt_ref)

    v = jnp.dot(x_ref[...], wt_ref[...], preferred_element_type=jnp.float32)
    s1_ref[...] += jnp.sum(v, axis=0, keepdims=True)
    s2_ref[...] += jnp.sum(v * v, axis=0, keepdims=True)

    ids = ids_ref[...].reshape(1, STATS_BLOCK)
    thr = (lax.broadcasted_iota(jnp.int32, (NUM_TILES, 1), 0) + 1) * SEGS_PER_TILE
    below = (ids < thr).astype(jnp.int32)
    cnt_ref[...] += jnp.sum(below, axis=1, keepdims=True).reshape(1, NUM_TILES)


def _stats_pass(inputs, ids, wt):
    nb = N // STATS_BLOCK
    return pl.pallas_call(
        _stats_body,
        grid=(nb,),
        in_specs=[
            pl.BlockSpec((STATS_BLOCK, IN_CH), lambda i: (i, 0)),
            pl.BlockSpec((STATS_BLOCK,), lambda i: (i,)),
            pl.BlockSpec((IN_CH, OUT_CH), lambda i: (0, 0)),
        ],
        out_specs=[
            pl.BlockSpec((1, OUT_CH), lambda i: (0, 0)),
            pl.BlockSpec((1, OUT_CH), lambda i: (0, 0)),
            pl.BlockSpec((1, NUM_TILES), lambda i: (0, 0)),
        ],
        out_shape=[
            jax.ShapeDtypeStruct((1, OUT_CH), jnp.float32),
            jax.ShapeDtypeStruct((1, OUT_CH), jnp.float32),
            jax.ShapeDtypeStruct((1, NUM_TILES), jnp.int32),
        ],
    )(inputs, ids, wt)


def _x_body(x_ref, wt_ref, b_ref, o_ref):
    v = jnp.dot(x_ref[...], wt_ref[...], preferred_element_type=jnp.float32)
    o_ref[...] = jnp.maximum(v + b_ref[...], 0.0)


def _x_pass(inputs, wt2, b2):
    nb = N // X_BLOCK
    return pl.pallas_call(
        _x_body,
        grid=(nb,),
        in_specs=[
            pl.BlockSpec((X_BLOCK, IN_CH), lambda i: (i, 0)),
            pl.BlockSpec((IN_CH, OUT_CH), lambda i: (0, 0)),
            pl.BlockSpec((1, OUT_CH), lambda i: (0, 0)),
        ],
        out_specs=pl.BlockSpec((X_BLOCK, OUT_CH), lambda i: (i, 0)),
        out_shape=jax.ShapeDtypeStruct((N, OUT_CH), jnp.float32),
    )(inputs, wt2, b2)


def _seg_body(x_hbm, ids_hbm, cnt_hbm, out_hbm, table,
              xb0, xb1, ib0, ib1, ob0, ob1, cntbuf,
              sx0, sx1, so0, so1):
    wid = lax.axis_index("s") * NUM_CORES + lax.axis_index("c")
    base = wid * SEGS_PER_TILE

    pltpu.sync_copy(cnt_hbm, cntbuf.at[0:NUM_TILES])
    r0 = jnp.where(
        wid == 0, 0, cntbuf[pl.ds(jnp.maximum(wid - 1, 0), HALF)][0])
    r1 = cntbuf[pl.ds(wid, HALF)][0]
    nrows = r1 - r0
    nch = (nrows + C - 1) // C

    def bases(lo):
        s8 = pl.multiple_of(jnp.minimum((lo // 8) * 8, N - (C + 8)), 8)
        sa = pl.multiple_of(jnp.minimum((lo // 16) * 16, N - IDS_SZ), 16)
        return s8, sa

    def stage_start(lo, xb, ib, sem):
        s8, sa = bases(lo)
        xo = pl.multiple_of(s8 * OUT_CH, 8 * OUT_CH)
        pltpu.make_async_copy(
            x_hbm.at[pl.ds(xo, XW)], xb.at[0:XW], sem).start()
        pltpu.make_async_copy(
            ids_hbm.at[pl.ds(sa, IDS_SZ)], ib.at[0:IDS_SZ], sem).start()

    def stage_wait(lo, xb, ib, sem):
        s8, sa = bases(lo)
        xo = pl.multiple_of(s8 * OUT_CH, 8 * OUT_CH)
        pltpu.make_async_copy(
            x_hbm.at[pl.ds(xo, XW)], xb.at[0:XW], sem).wait()
        pltpu.make_async_copy(
            ids_hbm.at[pl.ds(sa, IDS_SZ)], ib.at[0:IDS_SZ], sem).wait()

    neg = jnp.full((HALF,), -jnp.inf, dtype=jnp.float32)

    def a_lo(k):
        return r0 + k * C

    def a_process(k, xb, ib, carry):
        lo = a_lo(k)
        hi = jnp.minimum(lo + C, r1)
        s8, sa = bases(lo)

        def a_group(j, carry):
            cur, m0, m1 = carry
            g0 = lo + j * HALF
            iv = ib[pl.ds(g0 - sa, HALF)]
            xo0 = (g0 - s8) * OUT_CH
            for t in range(HALF):
                sid = iv[t]
                new = sid != cur
                x0 = xb[pl.ds(xo0 + t * OUT_CH, HALF)]
                x1 = xb[pl.ds(xo0 + t * OUT_CH + HALF, HALF)]
                m0 = jnp.where(new, x0, jnp.maximum(m0, x0))
                m1 = jnp.where(new, x1, jnp.maximum(m1, x1))
                to = (sid - base) * OUT_CH
                table[pl.ds(to, HALF)] = m0
                table[pl.ds(to + HALF, HALF)] = m1
                cur = sid
            return (cur, m0, m1)

        ng = (hi - lo) // HALF
        carry = lax.fori_loop(0, ng, a_group, carry)

        def a_row(g, carry):
            cur, m0, m1 = carry
            sid = ib[pl.ds(g - sa, HALF)][0]
            new = sid != cur
            xo = (g - s8) * OUT_CH
            x0 = xb[pl.ds(xo, HALF)]
            x1 = xb[pl.ds(xo + HALF, HALF)]
            m0 = jnp.where(new, x0, jnp.maximum(m0, x0))
            m1 = jnp.where(new, x1, jnp.maximum(m1, x1))
            to = (sid - base) * OUT_CH
            table[pl.ds(to, HALF)] = m0
            table[pl.ds(to + HALF, HALF)] = m1
            return (sid, m0, m1)

        return lax.fori_loop(lo + ng * HALF, hi, a_row, carry)

    @pl.when(nch > 0)
    def _phase_a():
        stage_start(a_lo(0), xb0, ib0, sx0)

        def a_pair(p, carry):
            k = p * 2

            stage_wait(a_lo(k), xb0, ib0, sx0)

            @pl.when(k + 1 < nch)
            def _():
                stage_start(a_lo(k + 1), xb1, ib1, sx1)

            carry2 = a_process(k, xb0, ib0, carry)

            def odd(c):
                stage_wait(a_lo(k + 1), xb1, ib1, sx1)

                @pl.when(k + 2 < nch)
                def _():
                    stage_start(a_lo(k + 2), xb0, ib0, sx0)

                return a_process(k + 1, xb1, ib1, c)

            return lax.fori_loop(
                0, jnp.where(k + 1 < nch, 1, 0), lambda _, c: odd(c), carry2)

        lax.fori_loop(0, (nch + 1) // 2, a_pair, (jnp.int32(-1), neg, neg))

    nfull = nrows // C
    rem = nrows - nfull * C
    nb_chunks = jnp.where(rem > 0, nfull + 1, nfull)

    def b_lo(k):
        return jnp.minimum(r0 + k * C, r1 - C)

    def b_process(k, xb, ib, ob):
        lo = b_lo(k)
        s8, sa = bases(lo)

        def b_group(j, _):
            g0 = lo + j * HALF
            iv = ib[pl.ds(g0 - sa, HALF)]
            xo0 = (g0 - s8) * OUT_CH
            oo0 = (g0 - lo) * OUT_CH
            for t in range(HALF):
                sid = iv[t]
                to = (sid - base) * OUT_CH
                t0 = table[pl.ds(to, HALF)]
                t1 = table[pl.ds(to + HALF, HALF)]
                o0 = (xb[pl.ds(xo0 + t * OUT_CH, HALF)] + t0) * 0.5
                o1 = (xb[pl.ds(xo0 + t * OUT_CH + HALF, HALF)] + t1) * 0.5
                ob[pl.ds(oo0 + t * OUT_CH, HALF)] = o0
                ob[pl.ds(oo0 + t * OUT_CH + HALF, HALF)] = o1
            return 0

        lax.fori_loop(0, C // HALF, b_group, 0)

    def b_write_start(k, ob, sem):
        o0 = pl.multiple_of(b_lo(k) * OUT_CH, OUT_CH)
        pltpu.make_async_copy(
            ob, out_hbm.at[pl.ds(o0, C * OUT_CH)], sem).start()

    def b_write_wait(k, ob, sem):
        o0 = pl.multiple_of(b_lo(k) * OUT_CH, OUT_CH)
        pltpu.make_async_copy(
            ob, out_hbm.at[pl.ds(o0, C * OUT_CH)], sem).wait()

    @pl.when(nrows >= C)
    def _b_main():
        stage_start(b_lo(0), xb0, ib0, sx0)

        def b_pair(p, _):
            k = p * 2

            stage_wait(b_lo(k), xb0, ib0, sx0)

            @pl.when(k + 1 < nb_chunks)
            def _():
                stage_start(b_lo(k + 1), xb1, ib1, sx1)

            @pl.when(k >= 2)
            def _():
                b_write_wait(k - 2, ob0, so0)

            b_process(k, xb0, ib0, ob0)
            b_write_start(k, ob0, so0)

            @pl.when(k + 1 < nb_chunks)
            def _odd():
                stage_wait(b_lo(k + 1), xb1, ib1, sx1)

                @pl.when(k + 2 < nb_chunks)
                def _():
                    stage_start(b_lo(k + 2), xb0, ib0, sx0)

                @pl.when(k >= 1)
                def _():
                    b_write_wait(k - 1, ob1, so1)

                b_process(k + 1, xb1, ib1, ob1)
                b_write_start(k + 1, ob1, so1)

            return 0

        lax.fori_loop(0, (nb_chunks + 1) // 2, b_pair, 0)

        last = nb_chunks - 1

        @pl.when(last % 2 == 0)
        def _():
            b_write_wait(last, ob0, so0)

        @pl.when((last >= 1) & (last % 2 == 1))
        def _():
            b_write_wait(last, ob1, so1)

        @pl.when((last >= 1) & (last % 2 == 0))
        def _():
            b_write_wait(last - 1, ob1, so1)

        @pl.when((last >= 2) & (last % 2 == 1))
        def _():
            b_write_wait(last - 1, ob0, so0)

    @pl.when((nrows > 0) & (nrows < C))
    def _b_small():
        s8, sa = bases(r0)
        stage_start(r0, xb0, ib0, sx0)
        stage_wait(r0, xb0, ib0, sx0)

        def row(g, c):
            sid = ib0[pl.ds(g - sa, HALF)][0]
            to = (sid - base) * OUT_CH
            t0 = table[pl.ds(to, HALF)]
            t1 = table[pl.ds(to + HALF, HALF)]
            xo = (g - s8) * OUT_CH
            oo = pl.multiple_of((g - r0) * OUT_CH, OUT_CH)
            ob0[pl.ds(oo, HALF)] = (xb0[pl.ds(xo, HALF)] + t0) * 0.5
            ob0[pl.ds(oo + HALF, HALF)] = (xb0[pl.ds(xo + HALF, HALF)] + t1) * 0.5
            dst = pl.multiple_of(g * OUT_CH, OUT_CH)
            pltpu.sync_copy(ob0.at[pl.ds(oo, OUT_CH)],
                            out_hbm.at[pl.ds(dst, OUT_CH)])
            return c

        lax.fori_loop(r0, r1, row, 0)


@functools.partial(
    pl.kernel,
    out_type=jax.ShapeDtypeStruct((N * OUT_CH,), jnp.float32),
    mesh=plsc.VectorSubcoreMesh(
        core_axis_name="c", subcore_axis_name="s", num_cores=NUM_CORES,
        num_subcores=NUM_SUBCORES),
    scratch_types=[
        pltpu.VMEM((SEGS_PER_TILE * OUT_CH,), jnp.float32),
        pltpu.VMEM((XW,), jnp.float32),
        pltpu.VMEM((XW,), jnp.float32),
        pltpu.VMEM((IDS_SZ + HALF,), jnp.int32),
        pltpu.VMEM((IDS_SZ + HALF,), jnp.int32),
        pltpu.VMEM((C * OUT_CH,), jnp.float32),
        pltpu.VMEM((C * OUT_CH,), jnp.float32),
        pltpu.VMEM((NUM_TILES + HALF,), jnp.int32),
        pltpu.SemaphoreType.DMA,
        pltpu.SemaphoreType.DMA,
        pltpu.SemaphoreType.DMA,
        pltpu.SemaphoreType.DMA,
    ],
)
def _seg_kernel(x_hbm, ids_hbm, cnt_hbm, out_hbm, table,
                xb0, xb1, ib0, ib1, ob0, ob1, cntbuf, sx0, sx1, so0, so1):
    _seg_body(x_hbm, ids_hbm, cnt_hbm, out_hbm, table,
              xb0, xb1, ib0, ib1, ob0, ob1, cntbuf, sx0, sx1, so0, so1)


def kernel(inputs, unq_inv, W, b, gamma, beta):
    ids = unq_inv.astype(jnp.int32)
    wt = W.T

    s1, s2, cnt = _stats_pass(inputs, ids, wt)
    s1 = s1[0]
    s2 = s2[0]

    mean = s1 / N + b
    var = s2 / N - (s1 / N) ** 2
    a = gamma * lax.rsqrt(var + EPS)
    d = beta - mean * a
    wt2 = wt * a[None, :]
    b2 = (b * a + d).reshape(1, OUT_CH)

    x = _x_pass(inputs, wt2, b2)

    out_flat = _seg_kernel(x.reshape(N * OUT_CH), ids, cnt[0])
    return out_flat.reshape(N, OUT_CH)


# speedup vs baseline: 5.2099x; 1.3010x over previous
import functools

import jax
import jax.numpy as jnp
from jax import lax
from jax.experimental import pallas as pl
from jax.experimental.pallas import tpu as pltpu
from jax.experimental.pallas import tpu_sc as plsc

N = 3200000
IN_CH = 10
OUT_CH = 32
NUM_SEGMENTS = 100000
EPS = 1e-3
NUM_CORES = 2
NUM_SUBCORES = 16
NUM_TILES = NUM_CORES * NUM_SUBCORES
SEGS_PER_TILE = NUM_SEGMENTS // NUM_TILES

STATS_BLOCK = 2000
X_BLOCK = 1000
NW = 64
WIDE = NW * IN_CH
XOUT = NW * OUT_CH
C = 128
IDS_SZ = C + 16
HALF = 16
XW = (C + 8) * OUT_CH


def _stats_body(x_ref, ids_ref, s_ref, g_ref, cnt_ref):
    i = pl.program_id(0)

    @pl.when(i == 0)
    def _init():
        s_ref[...] = jnp.zeros_like(s_ref)
        g_ref[...] = jnp.zeros_like(g_ref)
        cnt_ref[...] = jnp.zeros_like(cnt_ref)

    blk = x_ref[...]
    s_ref[...] += jnp.sum(blk, axis=0, keepdims=True)
    g_ref[...] += lax.dot_general(
        blk, blk, (((0,), (0,)), ((), ())),
        preferred_element_type=jnp.float32)

    ids = ids_ref[...].reshape(1, STATS_BLOCK * NW)
    thr = (lax.broadcasted_iota(jnp.int32, (NUM_TILES, 1), 0) + 1) * SEGS_PER_TILE
    below = (ids < thr).astype(jnp.int32)
    cnt_ref[...] += jnp.sum(below, axis=1, keepdims=True).reshape(1, NUM_TILES)


def _stats_pass(inputs_w, ids):
    nb = (N // NW) // STATS_BLOCK
    return pl.pallas_call(
        _stats_body,
        grid=(nb,),
        in_specs=[
            pl.BlockSpec((STATS_BLOCK, WIDE), lambda i: (i, 0)),
            pl.BlockSpec((STATS_BLOCK * NW,), lambda i: (i,)),
        ],
        out_specs=[
            pl.BlockSpec((1, WIDE), lambda i: (0, 0)),
            pl.BlockSpec((WIDE, WIDE), lambda i: (0, 0)),
            pl.BlockSpec((1, NUM_TILES), lambda i: (0, 0)),
        ],
        out_shape=[
            jax.ShapeDtypeStruct((1, WIDE), jnp.float32),
            jax.ShapeDtypeStruct((WIDE, WIDE), jnp.float32),
            jax.ShapeDtypeStruct((1, NUM_TILES), jnp.int32),
        ],
    )(inputs_w, ids)


def _x_body(x_ref, k4_ref, b4_ref, o_ref):
    k4 = k4_ref[...]
    b4 = b4_ref[...]
    for m in range(HALF):
        piece = jnp.dot(x_ref[:, 40 * m:40 * m + 40], k4,
                        preferred_element_type=jnp.float32)
        o_ref[:, 128 * m:128 * (m + 1)] = jnp.maximum(piece + b4, 0.0)


def _x_pass(inputs_w, k4, b4):
    nb = (N // NW) // X_BLOCK
    return pl.pallas_call(
        _x_body,
        grid=(nb,),
        in_specs=[
            pl.BlockSpec((X_BLOCK, WIDE), lambda i: (i, 0)),
            pl.BlockSpec((4 * IN_CH, 128), lambda i: (0, 0)),
            pl.BlockSpec((1, 128), lambda i: (0, 0)),
        ],
        out_specs=pl.BlockSpec((X_BLOCK, XOUT), lambda i: (i, 0)),
        out_shape=jax.ShapeDtypeStruct((N // NW, XOUT), jnp.float32),
    )(inputs_w, k4, b4)


def _seg_body(x_hbm, ids_hbm, cnt_hbm, out_hbm, table,
              xb0, xb1, ib0, ib1, ob0, ob1, cntbuf,
              sx0, sx1, so0, so1):
    wid = lax.axis_index("s") * NUM_CORES + lax.axis_index("c")
    base = wid * SEGS_PER_TILE

    pltpu.sync_copy(cnt_hbm, cntbuf.at[0:NUM_TILES])
    r0 = jnp.where(
        wid == 0, 0, cntbuf[pl.ds(jnp.maximum(wid - 1, 0), HALF)][0])
    r1 = cntbuf[pl.ds(wid, HALF)][0]
    nrows = r1 - r0
    nch = (nrows + C - 1) // C

    def bases(lo):
        s8 = pl.multiple_of(jnp.minimum((lo // 8) * 8, N - (C + 8)), 8)
        sa = pl.multiple_of(jnp.minimum((lo // 16) * 16, N - IDS_SZ), 16)
        return s8, sa

    def stage_start(lo, xb, ib, sem):
        s8, sa = bases(lo)
        xo = pl.multiple_of(s8 * OUT_CH, 8 * OUT_CH)
        pltpu.make_async_copy(
            x_hbm.at[pl.ds(xo, XW)], xb.at[0:XW], sem).start()
        pltpu.make_async_copy(
            ids_hbm.at[pl.ds(sa, IDS_SZ)], ib.at[0:IDS_SZ], sem).start()

    def stage_wait(lo, xb, ib, sem):
        s8, sa = bases(lo)
        xo = pl.multiple_of(s8 * OUT_CH, 8 * OUT_CH)
        pltpu.make_async_copy(
            x_hbm.at[pl.ds(xo, XW)], xb.at[0:XW], sem).wait()
        pltpu.make_async_copy(
            ids_hbm.at[pl.ds(sa, IDS_SZ)], ib.at[0:IDS_SZ], sem).wait()

    neg = jnp.full((HALF,), -jnp.inf, dtype=jnp.float32)

    def a_lo(k):
        return r0 + k * C

    def a_process(k, xb, ib, carry):
        lo = a_lo(k)
        hi = jnp.minimum(lo + C, r1)
        s8, sa = bases(lo)

        def a_group(j, carry):
            cur, m0, m1 = carry
            g0 = lo + j * HALF
            iv = ib[pl.ds(g0 - sa, HALF)]
            xo0 = (g0 - s8) * OUT_CH
            for t in range(HALF):
                sid = iv[t]
                new = sid != cur
                x0 = xb[pl.ds(xo0 + t * OUT_CH, HALF)]
                x1 = xb[pl.ds(xo0 + t * OUT_CH + HALF, HALF)]
                m0 = jnp.where(new, x0, jnp.maximum(m0, x0))
                m1 = jnp.where(new, x1, jnp.maximum(m1, x1))
                to = (sid - base) * OUT_CH
                table[pl.ds(to, HALF)] = m0
                table[pl.ds(to + HALF, HALF)] = m1
                cur = sid
            return (cur, m0, m1)

        ng = (hi - lo) // HALF
        carry = lax.fori_loop(0, ng, a_group, carry)

        def a_row(g, carry):
            cur, m0, m1 = carry
            sid = ib[pl.ds(g - sa, HALF)][0]
            new = sid != cur
            xo = (g - s8) * OUT_CH
            x0 = xb[pl.ds(xo, HALF)]
            x1 = xb[pl.ds(xo + HALF, HALF)]
            m0 = jnp.where(new, x0, jnp.maximum(m0, x0))
            m1 = jnp.where(new, x1, jnp.maximum(m1, x1))
            to = (sid - base) * OUT_CH
            table[pl.ds(to, HALF)] = m0
            table[pl.ds(to + HALF, HALF)] = m1
            return (sid, m0, m1)

        return lax.fori_loop(lo + ng * HALF, hi, a_row, carry)

    @pl.when(nch > 0)
    def _phase_a():
        stage_start(a_lo(0), xb0, ib0, sx0)

        def a_pair(p, carry):
            k = p * 2

            stage_wait(a_lo(k), xb0, ib0, sx0)

            @pl.when(k + 1 < nch)
            def _():
                stage_start(a_lo(k + 1), xb1, ib1, sx1)

            carry2 = a_process(k, xb0, ib0, carry)

            def odd(c):
                stage_wait(a_lo(k + 1), xb1, ib1, sx1)

                @pl.when(k + 2 < nch)
                def _():
                    stage_start(a_lo(k + 2), xb0, ib0, sx0)

                return a_process(k + 1, xb1, ib1, c)

            return lax.fori_loop(
                0, jnp.where(k + 1 < nch, 1, 0), lambda _, c: odd(c), carry2)

        lax.fori_loop(0, (nch + 1) // 2, a_pair, (jnp.int32(-1), neg, neg))

    nfull = nrows // C
    rem = nrows - nfull * C
    nb_chunks = jnp.where(rem > 0, nfull + 1, nfull)

    def b_lo(k):
        return jnp.minimum(r0 + k * C, r1 - C)

    def b_process(k, xb, ib, ob):
        lo = b_lo(k)
        s8, sa = bases(lo)

        def b_group(j, _):
            g0 = lo + j * HALF
            iv = ib[pl.ds(g0 - sa, HALF)]
            xo0 = (g0 - s8) * OUT_CH
            oo0 = (g0 - lo) * OUT_CH
            for t in range(HALF):
                sid = iv[t]
                to = (sid - base) * OUT_CH
                t0 = table[pl.ds(to, HALF)]
                t1 = table[pl.ds(to + HALF, HALF)]
                o0 = (xb[pl.ds(xo0 + t * OUT_CH, HALF)] + t0) * 0.5
                o1 = (xb[pl.ds(xo0 + t * OUT_CH + HALF, HALF)] + t1) * 0.5
                ob[pl.ds(oo0 + t * OUT_CH, HALF)] = o0
                ob[pl.ds(oo0 + t * OUT_CH + HALF, HALF)] = o1
            return 0

        lax.fori_loop(0, C // HALF, b_group, 0)

    def b_write_start(k, ob, sem):
        o0 = pl.multiple_of(b_lo(k) * OUT_CH, OUT_CH)
        pltpu.make_async_copy(
            ob, out_hbm.at[pl.ds(o0, C * OUT_CH)], sem).start()

    def b_write_wait(k, ob, sem):
        o0 = pl.multiple_of(b_lo(k) * OUT_CH, OUT_CH)
        pltpu.make_async_copy(
            ob, out_hbm.at[pl.ds(o0, C * OUT_CH)], sem).wait()

    @pl.when(nrows >= C)
    def _b_main():
        stage_start(b_lo(0), xb0, ib0, sx0)

        def b_pair(p, _):
            k = p * 2

            stage_wait(b_lo(k), xb0, ib0, sx0)

            @pl.when(k + 1 < nb_chunks)
            def _():
                stage_start(b_lo(k + 1), xb1, ib1, sx1)

            @pl.when(k >= 2)
            def _():
                b_write_wait(k - 2, ob0, so0)

            b_process(k, xb0, ib0, ob0)
            b_write_start(k, ob0, so0)

            @pl.when(k + 1 < nb_chunks)
            def _odd():
                stage_wait(b_lo(k + 1), xb1, ib1, sx1)

                @pl.when(k + 2 < nb_chunks)
                def _():
                    stage_start(b_lo(k + 2), xb0, ib0, sx0)

                @pl.when(k >= 1)
                def _():
                    b_write_wait(k - 1, ob1, so1)

                b_process(k + 1, xb1, ib1, ob1)
                b_write_start(k + 1, ob1, so1)

            return 0

        lax.fori_loop(0, (nb_chunks + 1) // 2, b_pair, 0)

        last = nb_chunks - 1

        @pl.when(last % 2 == 0)
        def _():
            b_write_wait(last, ob0, so0)

        @pl.when((last >= 1) & (last % 2 == 1))
        def _():
            b_write_wait(last, ob1, so1)

        @pl.when((last >= 1) & (last % 2 == 0))
        def _():
            b_write_wait(last - 1, ob1, so1)

        @pl.when((last >= 2) & (last % 2 == 1))
        def _():
            b_write_wait(last - 1, ob0, so0)

    @pl.when((nrows > 0) & (nrows < C))
    def _b_small():
        s8, sa = bases(r0)
        stage_start(r0, xb0, ib0, sx0)
        stage_wait(r0, xb0, ib0, sx0)

        def row(g, c):
            sid = ib0[pl.ds(g - sa, HALF)][0]
            to = (sid - base) * OUT_CH
            t0 = table[pl.ds(to, HALF)]
            t1 = table[pl.ds(to + HALF, HALF)]
            xo = (g - s8) * OUT_CH
            oo = pl.multiple_of((g - r0) * OUT_CH, OUT_CH)
            ob0[pl.ds(oo, HALF)] = (xb0[pl.ds(xo, HALF)] + t0) * 0.5
            ob0[pl.ds(oo + HALF, HALF)] = (xb0[pl.ds(xo + HALF, HALF)] + t1) * 0.5
            dst = pl.multiple_of(g * OUT_CH, OUT_CH)
            pltpu.sync_copy(ob0.at[pl.ds(oo, OUT_CH)],
                            out_hbm.at[pl.ds(dst, OUT_CH)])
            return c

        lax.fori_loop(r0, r1, row, 0)


@functools.partial(
    pl.kernel,
    out_type=jax.ShapeDtypeStruct((N * OUT_CH,), jnp.float32),
    mesh=plsc.VectorSubcoreMesh(
        core_axis_name="c", subcore_axis_name="s", num_cores=NUM_CORES,
        num_subcores=NUM_SUBCORES),
    scratch_types=[
        pltpu.VMEM((SEGS_PER_TILE * OUT_CH,), jnp.float32),
        pltpu.VMEM((XW,), jnp.float32),
        pltpu.VMEM((XW,), jnp.float32),
        pltpu.VMEM((IDS_SZ + HALF,), jnp.int32),
        pltpu.VMEM((IDS_SZ + HALF,), jnp.int32),
        pltpu.VMEM((C * OUT_CH,), jnp.float32),
        pltpu.VMEM((C * OUT_CH,), jnp.float32),
        pltpu.VMEM((NUM_TILES + HALF,), jnp.int32),
        pltpu.SemaphoreType.DMA,
        pltpu.SemaphoreType.DMA,
        pltpu.SemaphoreType.DMA,
        pltpu.SemaphoreType.DMA,
    ],
)
def _seg_kernel(x_hbm, ids_hbm, cnt_hbm, out_hbm, table,
                xb0, xb1, ib0, ib1, ob0, ob1, cntbuf, sx0, sx1, so0, so1):
    _seg_body(x_hbm, ids_hbm, cnt_hbm, out_hbm, table,
              xb0, xb1, ib0, ib1, ob0, ob1, cntbuf, sx0, sx1, so0, so1)


def kernel(inputs, unq_inv, W, b, gamma, beta):
    ids = unq_inv.astype(jnp.int32)
    wt = W.T
    inputs_w = inputs.reshape(N // NW, WIDE)

    s640, g640, cnt = _stats_pass(inputs_w, ids)
    sx = s640.reshape(NW, IN_CH).sum(axis=0)
    g10 = g640.reshape(NW, IN_CH, NW, IN_CH)
    g10 = jnp.einsum("jkjl->kl", g10)

    mean = sx @ wt / N + b
    ev2 = jnp.sum((g10 @ wt) * wt, axis=0) / N
    var = ev2 - (sx @ wt / N) ** 2
    a = gamma * lax.rsqrt(var + EPS)
    d = beta - mean * a
    wt2 = wt * a[None, :]
    b2 = b * a + d

    k4 = jnp.zeros((4 * IN_CH, 128), jnp.float32)
    for u in range(4):
        k4 = k4.at[IN_CH * u:IN_CH * (u + 1),
                   OUT_CH * u:OUT_CH * (u + 1)].set(wt2)
    b4 = jnp.tile(b2, 4).reshape(1, 128)

    x = _x_pass(inputs_w, k4, b4)

    out_flat = _seg_kernel(x.reshape(N * OUT_CH), ids, cnt[0])
    return out_flat.reshape(N, OUT_CH)


# speedup vs baseline: 5.4031x; 1.0371x over previous
import functools

import jax
import jax.numpy as jnp
from jax import lax
from jax.experimental import pallas as pl
from jax.experimental.pallas import tpu as pltpu
from jax.experimental.pallas import tpu_sc as plsc

N = 3200000
IN_CH = 10
OUT_CH = 32
NUM_SEGMENTS = 100000
EPS = 1e-3
NUM_CORES = 2
NUM_SUBCORES = 16
NUM_TILES = NUM_CORES * NUM_SUBCORES
SEGS_PER_TILE = NUM_SEGMENTS // NUM_TILES

STATS_BLOCK = 2000
X_BLOCK = 1000
NW = 64
WIDE = NW * IN_CH
XOUT = NW * OUT_CH
C = 128
IDS_SZ = C + 16
HALF = 16
XW = (C + 8) * OUT_CH


def _stats_body(x_ref, ids_ref, s_ref, g_ref, cnt_ref):
    i = pl.program_id(0)

    @pl.when(i == 0)
    def _init():
        s_ref[...] = jnp.zeros_like(s_ref)
        g_ref[...] = jnp.zeros_like(g_ref)
        cnt_ref[...] = jnp.zeros_like(cnt_ref)

    blk = x_ref[...]
    s_ref[...] += jnp.sum(blk, axis=0, keepdims=True)
    g_ref[...] += lax.dot_general(
        blk, blk, (((0,), (0,)), ((), ())),
        preferred_element_type=jnp.float32)

    ids = ids_ref[...].reshape(1, STATS_BLOCK * NW)
    thr = (lax.broadcasted_iota(jnp.int32, (NUM_TILES, 1), 0) + 1) * SEGS_PER_TILE
    below = (ids < thr).astype(jnp.int32)
    cnt_ref[...] += jnp.sum(below, axis=1, keepdims=True).reshape(1, NUM_TILES)


def _stats_pass(inputs_w, ids):
    nb = (N // NW) // STATS_BLOCK
    return pl.pallas_call(
        _stats_body,
        grid=(nb,),
        in_specs=[
            pl.BlockSpec((STATS_BLOCK, WIDE), lambda i: (i, 0)),
            pl.BlockSpec((STATS_BLOCK * NW,), lambda i: (i,)),
        ],
        out_specs=[
            pl.BlockSpec((1, WIDE), lambda i: (0, 0)),
            pl.BlockSpec((WIDE, WIDE), lambda i: (0, 0)),
            pl.BlockSpec((1, NUM_TILES), lambda i: (0, 0)),
        ],
        out_shape=[
            jax.ShapeDtypeStruct((1, WIDE), jnp.float32),
            jax.ShapeDtypeStruct((WIDE, WIDE), jnp.float32),
            jax.ShapeDtypeStruct((1, NUM_TILES), jnp.int32),
        ],
    )(inputs_w, ids)


def _x_body(x_ref, k4_ref, b4_ref, o_ref):
    k4 = k4_ref[...]
    b4 = b4_ref[...]
    for m in range(HALF):
        piece = jnp.dot(x_ref[:, 40 * m:40 * m + 40], k4,
                        preferred_element_type=jnp.float32)
        o_ref[:, m, :] = jnp.maximum(piece + b4, 0.0)


def _x_pass(inputs_w, k4, b4):
    nb = (N // NW) // X_BLOCK
    return pl.pallas_call(
        _x_body,
        grid=(nb,),
        in_specs=[
            pl.BlockSpec((X_BLOCK, WIDE), lambda i: (i, 0)),
            pl.BlockSpec((4 * IN_CH, 128), lambda i: (0, 0)),
            pl.BlockSpec((1, 128), lambda i: (0, 0)),
        ],
        out_specs=pl.BlockSpec((X_BLOCK, HALF, 128), lambda i: (i, 0, 0)),
        out_shape=jax.ShapeDtypeStruct((N // NW, HALF, 128), jnp.float32),
    )(inputs_w, k4, b4)


def _seg_body(x_hbm, ids_hbm, cnt_hbm, out_hbm, table,
              xb0, xb1, ib0, ib1, ob0, ob1, cntbuf,
              sx0, sx1, so0, so1):
    wid = lax.axis_index("s") * NUM_CORES + lax.axis_index("c")
    base = wid * SEGS_PER_TILE

    pltpu.sync_copy(cnt_hbm, cntbuf.at[0:NUM_TILES])
    r0 = jnp.where(
        wid == 0, 0, cntbuf[pl.ds(jnp.maximum(wid - 1, 0), HALF)][0])
    r1 = cntbuf[pl.ds(wid, HALF)][0]
    nrows = r1 - r0
    nch = (nrows + C - 1) // C

    def bases(lo):
        s8 = pl.multiple_of(jnp.minimum((lo // 8) * 8, N - (C + 8)), 8)
        sa = pl.multiple_of(jnp.minimum((lo // 16) * 16, N - IDS_SZ), 16)
        return s8, sa

    def stage_start(lo, xb, ib, sem):
        s8, sa = bases(lo)
        xo = pl.multiple_of(s8 * OUT_CH, 8 * OUT_CH)
        pltpu.make_async_copy(
            x_hbm.at[pl.ds(xo, XW)], xb.at[0:XW], sem).start()
        pltpu.make_async_copy(
            ids_hbm.at[pl.ds(sa, IDS_SZ)], ib.at[0:IDS_SZ], sem).start()

    def stage_wait(lo, xb, ib, sem):
        s8, sa = bases(lo)
        xo = pl.multiple_of(s8 * OUT_CH, 8 * OUT_CH)
        pltpu.make_async_copy(
            x_hbm.at[pl.ds(xo, XW)], xb.at[0:XW], sem).wait()
        pltpu.make_async_copy(
            ids_hbm.at[pl.ds(sa, IDS_SZ)], ib.at[0:IDS_SZ], sem).wait()

    neg = jnp.full((HALF,), -jnp.inf, dtype=jnp.float32)

    def a_lo(k):
        return r0 + k * C

    def a_process(k, xb, ib, carry):
        lo = a_lo(k)
        hi = jnp.minimum(lo + C, r1)
        s8, sa = bases(lo)

        def a_group(j, carry):
            cur, m0, m1 = carry
            g0 = lo + j * HALF
            iv = ib[pl.ds(g0 - sa, HALF)]
            xo0 = (g0 - s8) * OUT_CH
            for t in range(HALF):
                sid = iv[t]
                new = sid != cur
                x0 = xb[pl.ds(xo0 + t * OUT_CH, HALF)]
                x1 = xb[pl.ds(xo0 + t * OUT_CH + HALF, HALF)]
                m0 = jnp.where(new, x0, jnp.maximum(m0, x0))
                m1 = jnp.where(new, x1, jnp.maximum(m1, x1))
                to = (sid - base) * OUT_CH
                table[pl.ds(to, HALF)] = m0
                table[pl.ds(to + HALF, HALF)] = m1
                cur = sid
            return (cur, m0, m1)

        ng = (hi - lo) // HALF
        carry = lax.fori_loop(0, ng, a_group, carry)

        def a_row(g, carry):
            cur, m0, m1 = carry
            sid = ib[pl.ds(g - sa, HALF)][0]
            new = sid != cur
            xo = (g - s8) * OUT_CH
            x0 = xb[pl.ds(xo, HALF)]
            x1 = xb[pl.ds(xo + HALF, HALF)]
            m0 = jnp.where(new, x0, jnp.maximum(m0, x0))
            m1 = jnp.where(new, x1, jnp.maximum(m1, x1))
            to = (sid - base) * OUT_CH
            table[pl.ds(to, HALF)] = m0
            table[pl.ds(to + HALF, HALF)] = m1
            return (sid, m0, m1)

        return lax.fori_loop(lo + ng * HALF, hi, a_row, carry)

    @pl.when(nch > 0)
    def _phase_a():
        stage_start(a_lo(0), xb0, ib0, sx0)

        def a_pair(p, carry):
            k = p * 2

            stage_wait(a_lo(k), xb0, ib0, sx0)

            @pl.when(k + 1 < nch)
            def _():
                stage_start(a_lo(k + 1), xb1, ib1, sx1)

            carry2 = a_process(k, xb0, ib0, carry)

            def odd(c):
                stage_wait(a_lo(k + 1), xb1, ib1, sx1)

                @pl.when(k + 2 < nch)
                def _():
                    stage_start(a_lo(k + 2), xb0, ib0, sx0)

                return a_process(k + 1, xb1, ib1, c)

            return lax.fori_loop(
                0, jnp.where(k + 1 < nch, 1, 0), lambda _, c: odd(c), carry2)

        lax.fori_loop(0, (nch + 1) // 2, a_pair, (jnp.int32(-1), neg, neg))

    nfull = nrows // C
    rem = nrows - nfull * C
    nb_chunks = jnp.where(rem > 0, nfull + 1, nfull)

    def b_lo(k):
        return jnp.minimum(r0 + k * C, r1 - C)

    def b_process(k, xb, ib, ob):
        lo = b_lo(k)
        s8, sa = bases(lo)

        def b_group(j, _):
            g0 = lo + j * HALF
            iv = ib[pl.ds(g0 - sa, HALF)]
            xo0 = (g0 - s8) * OUT_CH
            oo0 = (g0 - lo) * OUT_CH
            for t in range(HALF):
                sid = iv[t]
                to = (sid - base) * OUT_CH
                t0 = table[pl.ds(to, HALF)]
                t1 = table[pl.ds(to + HALF, HALF)]
                o0 = (xb[pl.ds(xo0 + t * OUT_CH, HALF)] + t0) * 0.5
                o1 = (xb[pl.ds(xo0 + t * OUT_CH + HALF, HALF)] + t1) * 0.5
                ob[pl.ds(oo0 + t * OUT_CH, HALF)] = o0
                ob[pl.ds(oo0 + t * OUT_CH + HALF, HALF)] = o1
            return 0

        lax.fori_loop(0, C // HALF, b_group, 0)

    def b_write_start(k, ob, sem):
        o0 = pl.multiple_of(b_lo(k) * OUT_CH, OUT_CH)
        pltpu.make_async_copy(
            ob, out_hbm.at[pl.ds(o0, C * OUT_CH)], sem).start()

    def b_write_wait(k, ob, sem):
        o0 = pl.multiple_of(b_lo(k) * OUT_CH, OUT_CH)
        pltpu.make_async_copy(
            ob, out_hbm.at[pl.ds(o0, C * OUT_CH)], sem).wait()

    @pl.when(nrows >= C)
    def _b_main():
        stage_start(b_lo(0), xb0, ib0, sx0)

        def b_pair(p, _):
            k = p * 2

            stage_wait(b_lo(k), xb0, ib0, sx0)

            @pl.when(k + 1 < nb_chunks)
            def _():
                stage_start(b_lo(k + 1), xb1, ib1, sx1)

            @pl.when(k >= 2)
            def _():
                b_write_wait(k - 2, ob0, so0)

            b_process(k, xb0, ib0, ob0)
            b_write_start(k, ob0, so0)

            @pl.when(k + 1 < nb_chunks)
            def _odd():
                stage_wait(b_lo(k + 1), xb1, ib1, sx1)

                @pl.when(k + 2 < nb_chunks)
                def _():
                    stage_start(b_lo(k + 2), xb0, ib0, sx0)

                @pl.when(k >= 1)
                def _():
                    b_write_wait(k - 1, ob1, so1)

                b_process(k + 1, xb1, ib1, ob1)
                b_write_start(k + 1, ob1, so1)

            return 0

        lax.fori_loop(0, (nb_chunks + 1) // 2, b_pair, 0)

        last = nb_chunks - 1

        @pl.when(last % 2 == 0)
        def _():
            b_write_wait(last, ob0, so0)

        @pl.when((last >= 1) & (last % 2 == 1))
        def _():
            b_write_wait(last, ob1, so1)

        @pl.when((last >= 1) & (last % 2 == 0))
        def _():
            b_write_wait(last - 1, ob1, so1)

        @pl.when((last >= 2) & (last % 2 == 1))
        def _():
            b_write_wait(last - 1, ob0, so0)

    @pl.when((nrows > 0) & (nrows < C))
    def _b_small():
        s8, sa = bases(r0)
        stage_start(r0, xb0, ib0, sx0)
        stage_wait(r0, xb0, ib0, sx0)

        def row(g, c):
            sid = ib0[pl.ds(g - sa, HALF)][0]
            to = (sid - base) * OUT_CH
            t0 = table[pl.ds(to, HALF)]
            t1 = table[pl.ds(to + HALF, HALF)]
            xo = (g - s8) * OUT_CH
            oo = pl.multiple_of((g - r0) * OUT_CH, OUT_CH)
            ob0[pl.ds(oo, HALF)] = (xb0[pl.ds(xo, HALF)] + t0) * 0.5
            ob0[pl.ds(oo + HALF, HALF)] = (xb0[pl.ds(xo + HALF, HALF)] + t1) * 0.5
            dst = pl.multiple_of(g * OUT_CH, OUT_CH)
            pltpu.sync_copy(ob0.at[pl.ds(oo, OUT_CH)],
                            out_hbm.at[pl.ds(dst, OUT_CH)])
            return c

        lax.fori_loop(r0, r1, row, 0)


@functools.partial(
    pl.kernel,
    out_type=jax.ShapeDtypeStruct((N * OUT_CH,), jnp.float32),
    mesh=plsc.VectorSubcoreMesh(
        core_axis_name="c", subcore_axis_name="s", num_cores=NUM_CORES,
        num_subcores=NUM_SUBCORES),
    scratch_types=[
        pltpu.VMEM((SEGS_PER_TILE * OUT_CH,), jnp.float32),
        pltpu.VMEM((XW,), jnp.float32),
        pltpu.VMEM((XW,), jnp.float32),
        pltpu.VMEM((IDS_SZ + HALF,), jnp.int32),
        pltpu.VMEM((IDS_SZ + HALF,), jnp.int32),
        pltpu.VMEM((C * OUT_CH,), jnp.float32),
        pltpu.VMEM((C * OUT_CH,), jnp.float32),
        pltpu.VMEM((NUM_TILES + HALF,), jnp.int32),
        pltpu.SemaphoreType.DMA,
        pltpu.SemaphoreType.DMA,
        pltpu.SemaphoreType.DMA,
        pltpu.SemaphoreType.DMA,
    ],
)
def _seg_kernel(x_hbm, ids_hbm, cnt_hbm, out_hbm, table,
                xb0, xb1, ib0, ib1, ob0, ob1, cntbuf, sx0, sx1, so0, so1):
    _seg_body(x_hbm, ids_hbm, cnt_hbm, out_hbm, table,
              xb0, xb1, ib0, ib1, ob0, ob1, cntbuf, sx0, sx1, so0, so1)


def kernel(inputs, unq_inv, W, b, gamma, beta):
    ids = unq_inv.astype(jnp.int32)
    wt = W.T
    inputs_w = inputs.reshape(N // NW, WIDE)

    s640, g640, cnt = _stats_pass(inputs_w, ids)
    sx = s640.reshape(NW, IN_CH).sum(axis=0)
    g10 = g640.reshape(NW, IN_CH, NW, IN_CH)
    g10 = jnp.einsum("jkjl->kl", g10)

    mean = sx @ wt / N + b
    ev2 = jnp.sum((g10 @ wt) * wt, axis=0) / N
    var = ev2 - (sx @ wt / N) ** 2
    a = gamma * lax.rsqrt(var + EPS)
    d = beta - mean * a
    wt2 = wt * a[None, :]
    b2 = b * a + d

    k4 = jnp.zeros((4 * IN_CH, 128), jnp.float32)
    for u in range(4):
        k4 = k4.at[IN_CH * u:IN_CH * (u + 1),
                   OUT_CH * u:OUT_CH * (u + 1)].set(wt2)
    b4 = jnp.tile(b2, 4).reshape(1, 128)

    x = _x_pass(inputs_w, k4, b4)

    out_flat = _seg_kernel(x.reshape(N * OUT_CH), ids, cnt[0])
    return out_flat.reshape(N, OUT_CH)


# speedup vs baseline: 5.4502x; 1.0087x over previous
import functools

import jax
import jax.numpy as jnp
from jax import lax
from jax.experimental import pallas as pl
from jax.experimental.pallas import tpu as pltpu
from jax.experimental.pallas import tpu_sc as plsc

N = 3200000
IN_CH = 10
OUT_CH = 32
NUM_SEGMENTS = 100000
EPS = 1e-3
NUM_CORES = 2
NUM_SUBCORES = 16
NUM_TILES = NUM_CORES * NUM_SUBCORES
SEGS_PER_TILE = NUM_SEGMENTS // NUM_TILES

STATS_BLOCK = 2000
X_BLOCK = 1000
NW = 64
WIDE = NW * IN_CH
XOUT = NW * OUT_CH
C = 192
IDS_SZ = C + 16
HALF = 16
XW = (C + 8) * OUT_CH


def _stats_body(x_ref, ids_ref, s_ref, g_ref, cnt_ref):
    i = pl.program_id(0)

    @pl.when(i == 0)
    def _init():
        s_ref[...] = jnp.zeros_like(s_ref)
        g_ref[...] = jnp.zeros_like(g_ref)
        cnt_ref[...] = jnp.zeros_like(cnt_ref)

    blk = x_ref[...]
    s_ref[...] += jnp.sum(blk, axis=0, keepdims=True)
    g_ref[...] += lax.dot_general(
        blk, blk, (((0,), (0,)), ((), ())),
        preferred_element_type=jnp.float32)

    ids = ids_ref[...].reshape(1, STATS_BLOCK * NW)
    thr = (lax.broadcasted_iota(jnp.int32, (NUM_TILES, 1), 0) + 1) * SEGS_PER_TILE
    below = (ids < thr).astype(jnp.int32)
    cnt_ref[...] += jnp.sum(below, axis=1, keepdims=True).reshape(1, NUM_TILES)


def _stats_pass(inputs_w, ids):
    nb = (N // NW) // STATS_BLOCK
    return pl.pallas_call(
        _stats_body,
        grid=(nb,),
        in_specs=[
            pl.BlockSpec((STATS_BLOCK, WIDE), lambda i: (i, 0)),
            pl.BlockSpec((STATS_BLOCK * NW,), lambda i: (i,)),
        ],
        out_specs=[
            pl.BlockSpec((1, WIDE), lambda i: (0, 0)),
            pl.BlockSpec((WIDE, WIDE), lambda i: (0, 0)),
            pl.BlockSpec((1, NUM_TILES), lambda i: (0, 0)),
        ],
        out_shape=[
            jax.ShapeDtypeStruct((1, WIDE), jnp.float32),
            jax.ShapeDtypeStruct((WIDE, WIDE), jnp.float32),
            jax.ShapeDtypeStruct((1, NUM_TILES), jnp.int32),
        ],
    )(inputs_w, ids)


def _x_body(x_ref, k4_ref, b4_ref, o_ref):
    k4 = k4_ref[...]
    b4 = b4_ref[...]
    for m in range(HALF):
        piece = jnp.dot(x_ref[:, 40 * m:40 * m + 40], k4,
                        preferred_element_type=jnp.float32)
        o_ref[:, m, :] = jnp.maximum(piece + b4, 0.0)


def _x_pass(inputs_w, k4, b4):
    nb = (N // NW) // X_BLOCK
    return pl.pallas_call(
        _x_body,
        grid=(nb,),
        in_specs=[
            pl.BlockSpec((X_BLOCK, WIDE), lambda i: (i, 0)),
            pl.BlockSpec((4 * IN_CH, 128), lambda i: (0, 0)),
            pl.BlockSpec((1, 128), lambda i: (0, 0)),
        ],
        out_specs=pl.BlockSpec((X_BLOCK, HALF, 128), lambda i: (i, 0, 0)),
        out_shape=jax.ShapeDtypeStruct((N // NW, HALF, 128), jnp.float32),
    )(inputs_w, k4, b4)


def _seg_body(x_hbm, ids_hbm, cnt_hbm, out_hbm, table,
              xb0, xb1, ib0, ib1, ob0, ob1, cntbuf,
              sx0, sx1, so0, so1):
    wid = lax.axis_index("s") * NUM_CORES + lax.axis_index("c")
    base = wid * SEGS_PER_TILE

    pltpu.sync_copy(cnt_hbm, cntbuf.at[0:NUM_TILES])
    r0 = jnp.where(
        wid == 0, 0, cntbuf[pl.ds(jnp.maximum(wid - 1, 0), HALF)][0])
    r1 = cntbuf[pl.ds(wid, HALF)][0]
    nrows = r1 - r0
    nch = (nrows + C - 1) // C

    def bases(lo):
        s8 = pl.multiple_of(jnp.minimum((lo // 8) * 8, N - (C + 8)), 8)
        sa = pl.multiple_of(jnp.minimum((lo // 16) * 16, N - IDS_SZ), 16)
        return s8, sa

    def stage_start(lo, xb, ib, sem):
        s8, sa = bases(lo)
        xo = pl.multiple_of(s8 * OUT_CH, 8 * OUT_CH)
        pltpu.make_async_copy(
            x_hbm.at[pl.ds(xo, XW)], xb.at[0:XW], sem).start()
        pltpu.make_async_copy(
            ids_hbm.at[pl.ds(sa, IDS_SZ)], ib.at[0:IDS_SZ], sem).start()

    def stage_wait(lo, xb, ib, sem):
        s8, sa = bases(lo)
        xo = pl.multiple_of(s8 * OUT_CH, 8 * OUT_CH)
        pltpu.make_async_copy(
            x_hbm.at[pl.ds(xo, XW)], xb.at[0:XW], sem).wait()
        pltpu.make_async_copy(
            ids_hbm.at[pl.ds(sa, IDS_SZ)], ib.at[0:IDS_SZ], sem).wait()

    neg = jnp.full((HALF,), -jnp.inf, dtype=jnp.float32)

    def a_lo(k):
        return r0 + k * C

    def a_process(k, xb, ib, carry):
        lo = a_lo(k)
        hi = jnp.minimum(lo + C, r1)
        s8, sa = bases(lo)

        def a_group(j, carry):
            cur, m0, m1 = carry
            g0 = lo + j * HALF
            iv = ib[pl.ds(g0 - sa, HALF)]
            xo0 = (g0 - s8) * OUT_CH
            for t in range(HALF):
                sid = iv[t]
                new = sid != cur
                x0 = xb[pl.ds(xo0 + t * OUT_CH, HALF)]
                x1 = xb[pl.ds(xo0 + t * OUT_CH + HALF, HALF)]
                m0 = jnp.where(new, x0, jnp.maximum(m0, x0))
                m1 = jnp.where(new, x1, jnp.maximum(m1, x1))
                to = (sid - base) * OUT_CH
                table[pl.ds(to, HALF)] = m0
                table[pl.ds(to + HALF, HALF)] = m1
                cur = sid
            return (cur, m0, m1)

        ng = (hi - lo) // HALF
        carry = lax.fori_loop(0, ng, a_group, carry)

        def a_row(g, carry):
            cur, m0, m1 = carry
            sid = ib[pl.ds(g - sa, HALF)][0]
            new = sid != cur
            xo = (g - s8) * OUT_CH
            x0 = xb[pl.ds(xo, HALF)]
            x1 = xb[pl.ds(xo + HALF, HALF)]
            m0 = jnp.where(new, x0, jnp.maximum(m0, x0))
            m1 = jnp.where(new, x1, jnp.maximum(m1, x1))
            to = (sid - base) * OUT_CH
            table[pl.ds(to, HALF)] = m0
            table[pl.ds(to + HALF, HALF)] = m1
            return (sid, m0, m1)

        return lax.fori_loop(lo + ng * HALF, hi, a_row, carry)

    @pl.when(nch > 0)
    def _phase_a():
        stage_start(a_lo(0), xb0, ib0, sx0)

        def a_pair(p, carry):
            k = p * 2

            stage_wait(a_lo(k), xb0, ib0, sx0)

            @pl.when(k + 1 < nch)
            def _():
                stage_start(a_lo(k + 1), xb1, ib1, sx1)

            carry2 = a_process(k, xb0, ib0, carry)

            def odd(c):
                stage_wait(a_lo(k + 1), xb1, ib1, sx1)

                @pl.when(k + 2 < nch)
                def _():
                    stage_start(a_lo(k + 2), xb0, ib0, sx0)

                return a_process(k + 1, xb1, ib1, c)

            return lax.fori_loop(
                0, jnp.where(k + 1 < nch, 1, 0), lambda _, c: odd(c), carry2)

        lax.fori_loop(0, (nch + 1) // 2, a_pair, (jnp.int32(-1), neg, neg))

    nfull = nrows // C
    rem = nrows - nfull * C
    nb_chunks = jnp.where(rem > 0, nfull + 1, nfull)

    def b_lo(k):
        return jnp.minimum(r0 + k * C, r1 - C)

    def b_process(k, xb, ib, ob):
        lo = b_lo(k)
        s8, sa = bases(lo)

        def b_group(j, _):
            g0 = lo + j * HALF
            iv = ib[pl.ds(g0 - sa, HALF)]
            xo0 = (g0 - s8) * OUT_CH
            oo0 = (g0 - lo) * OUT_CH
            for t in range(HALF):
                sid = iv[t]
                to = (sid - base) * OUT_CH
                t0 = table[pl.ds(to, HALF)]
                t1 = table[pl.ds(to + HALF, HALF)]
                o0 = (xb[pl.ds(xo0 + t * OUT_CH, HALF)] + t0) * 0.5
                o1 = (xb[pl.ds(xo0 + t * OUT_CH + HALF, HALF)] + t1) * 0.5
                ob[pl.ds(oo0 + t * OUT_CH, HALF)] = o0
                ob[pl.ds(oo0 + t * OUT_CH + HALF, HALF)] = o1
            return 0

        lax.fori_loop(0, C // HALF, b_group, 0)

    def b_write_start(k, ob, sem):
        o0 = pl.multiple_of(b_lo(k) * OUT_CH, OUT_CH)
        pltpu.make_async_copy(
            ob, out_hbm.at[pl.ds(o0, C * OUT_CH)], sem).start()

    def b_write_wait(k, ob, sem):
        o0 = pl.multiple_of(b_lo(k) * OUT_CH, OUT_CH)
        pltpu.make_async_copy(
            ob, out_hbm.at[pl.ds(o0, C * OUT_CH)], sem).wait()

    @pl.when(nrows >= C)
    def _b_main():
        stage_start(b_lo(0), xb0, ib0, sx0)

        def b_pair(p, _):
            k = p * 2

            stage_wait(b_lo(k), xb0, ib0, sx0)

            @pl.when(k + 1 < nb_chunks)
            def _():
                stage_start(b_lo(k + 1), xb1, ib1, sx1)

            @pl.when(k >= 2)
            def _():
                b_write_wait(k - 2, ob0, so0)

            b_process(k, xb0, ib0, ob0)
            b_write_start(k, ob0, so0)

            @pl.when(k + 1 < nb_chunks)
            def _odd():
                stage_wait(b_lo(k + 1), xb1, ib1, sx1)

                @pl.when(k + 2 < nb_chunks)
                def _():
                    stage_start(b_lo(k + 2), xb0, ib0, sx0)

                @pl.when(k >= 1)
                def _():
                    b_write_wait(k - 1, ob1, so1)

                b_process(k + 1, xb1, ib1, ob1)
                b_write_start(k + 1, ob1, so1)

            return 0

        lax.fori_loop(0, (nb_chunks + 1) // 2, b_pair, 0)

        last = nb_chunks - 1

        @pl.when(last % 2 == 0)
        def _():
            b_write_wait(last, ob0, so0)

        @pl.when((last >= 1) & (last % 2 == 1))
        def _():
            b_write_wait(last, ob1, so1)

        @pl.when((last >= 1) & (last % 2 == 0))
        def _():
            b_write_wait(last - 1, ob1, so1)

        @pl.when((last >= 2) & (last % 2 == 1))
        def _():
            b_write_wait(last - 1, ob0, so0)

    @pl.when((nrows > 0) & (nrows < C))
    def _b_small():
        s8, sa = bases(r0)
        stage_start(r0, xb0, ib0, sx0)
        stage_wait(r0, xb0, ib0, sx0)

        def row(g, c):
            sid = ib0[pl.ds(g - sa, HALF)][0]
            to = (sid - base) * OUT_CH
            t0 = table[pl.ds(to, HALF)]
            t1 = table[pl.ds(to + HALF, HALF)]
            xo = (g - s8) * OUT_CH
            oo = pl.multiple_of((g - r0) * OUT_CH, OUT_CH)
            ob0[pl.ds(oo, HALF)] = (xb0[pl.ds(xo, HALF)] + t0) * 0.5
            ob0[pl.ds(oo + HALF, HALF)] = (xb0[pl.ds(xo + HALF, HALF)] + t1) * 0.5
            dst = pl.multiple_of(g * OUT_CH, OUT_CH)
            pltpu.sync_copy(ob0.at[pl.ds(oo, OUT_CH)],
                            out_hbm.at[pl.ds(dst, OUT_CH)])
            return c

        lax.fori_loop(r0, r1, row, 0)


@functools.partial(
    pl.kernel,
    out_type=jax.ShapeDtypeStruct((N * OUT_CH,), jnp.float32),
    mesh=plsc.VectorSubcoreMesh(
        core_axis_name="c", subcore_axis_name="s", num_cores=NUM_CORES,
        num_subcores=NUM_SUBCORES),
    scratch_types=[
        pltpu.VMEM((SEGS_PER_TILE * OUT_CH,), jnp.float32),
        pltpu.VMEM((XW,), jnp.float32),
        pltpu.VMEM((XW,), jnp.float32),
        pltpu.VMEM((IDS_SZ + HALF,), jnp.int32),
        pltpu.VMEM((IDS_SZ + HALF,), jnp.int32),
        pltpu.VMEM((C * OUT_CH,), jnp.float32),
        pltpu.VMEM((C * OUT_CH,), jnp.float32),
        pltpu.VMEM((NUM_TILES + HALF,), jnp.int32),
        pltpu.SemaphoreType.DMA,
        pltpu.SemaphoreType.DMA,
        pltpu.SemaphoreType.DMA,
        pltpu.SemaphoreType.DMA,
    ],
)
def _seg_kernel(x_hbm, ids_hbm, cnt_hbm, out_hbm, table,
                xb0, xb1, ib0, ib1, ob0, ob1, cntbuf, sx0, sx1, so0, so1):
    _seg_body(x_hbm, ids_hbm, cnt_hbm, out_hbm, table,
              xb0, xb1, ib0, ib1, ob0, ob1, cntbuf, sx0, sx1, so0, so1)


def kernel(inputs, unq_inv, W, b, gamma, beta):
    ids = unq_inv.astype(jnp.int32)
    wt = W.T
    inputs_w = inputs.reshape(N // NW, WIDE)

    s640, g640, cnt = _stats_pass(inputs_w, ids)
    sx = s640.reshape(NW, IN_CH).sum(axis=0)
    g10 = g640.reshape(NW, IN_CH, NW, IN_CH)
    g10 = jnp.einsum("jkjl->kl", g10)

    mean = sx @ wt / N + b
    ev2 = jnp.sum((g10 @ wt) * wt, axis=0) / N
    var = ev2 - (sx @ wt / N) ** 2
    a = gamma * lax.rsqrt(var + EPS)
    d = beta - mean * a
    wt2 = wt * a[None, :]
    b2 = b * a + d

    k4 = jnp.zeros((4 * IN_CH, 128), jnp.float32)
    for u in range(4):
        k4 = k4.at[IN_CH * u:IN_CH * (u + 1),
                   OUT_CH * u:OUT_CH * (u + 1)].set(wt2)
    b4 = jnp.tile(b2, 4).reshape(1, 128)

    x = _x_pass(inputs_w, k4, b4)

    out_flat = _seg_kernel(x.reshape(N * OUT_CH), ids, cnt[0])
    return out_flat.reshape(N, OUT_CH)


# speedup vs baseline: 5.4920x; 1.0077x over previous
import functools

import jax
import jax.numpy as jnp
from jax import lax
from jax.experimental import pallas as pl
from jax.experimental.pallas import tpu as pltpu
from jax.experimental.pallas import tpu_sc as plsc

N = 3200000
IN_CH = 10
OUT_CH = 32
NUM_SEGMENTS = 100000
EPS = 1e-3
NUM_CORES = 2
NUM_SUBCORES = 16
NUM_TILES = NUM_CORES * NUM_SUBCORES
SEGS_PER_TILE = NUM_SEGMENTS // NUM_TILES

STATS_BLOCK = 2000
X_BLOCK = 1000
NW = 64
WIDE = NW * IN_CH
XOUT = NW * OUT_CH
C = 192
IDS_SZ = C + 16
HALF = 16
XW = (C + 8) * OUT_CH


def _stats_body(x_ref, ids_ref, s_ref, g_ref, cnt_ref):
    i = pl.program_id(0)

    @pl.when(i == 0)
    def _init():
        s_ref[...] = jnp.zeros_like(s_ref)
        g_ref[...] = jnp.zeros_like(g_ref)
        cnt_ref[...] = jnp.zeros_like(cnt_ref)

    blk = x_ref[...]
    s_ref[...] += jnp.sum(blk, axis=0, keepdims=True)
    g_ref[...] += lax.dot_general(
        blk, blk, (((0,), (0,)), ((), ())),
        preferred_element_type=jnp.float32)

    ids = ids_ref[...].reshape(1, STATS_BLOCK * NW)
    thr = (lax.broadcasted_iota(jnp.int32, (NUM_TILES, 1), 0) + 1) * SEGS_PER_TILE
    below = (ids < thr).astype(jnp.int32)
    cnt_ref[...] += jnp.sum(below, axis=1, keepdims=True).reshape(1, NUM_TILES)


def _stats_pass(inputs_w, ids):
    nb = (N // NW) // STATS_BLOCK
    return pl.pallas_call(
        _stats_body,
        grid=(nb,),
        in_specs=[
            pl.BlockSpec((STATS_BLOCK, WIDE), lambda i: (i, 0)),
            pl.BlockSpec((STATS_BLOCK * NW,), lambda i: (i,)),
        ],
        out_specs=[
            pl.BlockSpec((1, WIDE), lambda i: (0, 0)),
            pl.BlockSpec((WIDE, WIDE), lambda i: (0, 0)),
            pl.BlockSpec((1, NUM_TILES), lambda i: (0, 0)),
        ],
        out_shape=[
            jax.ShapeDtypeStruct((1, WIDE), jnp.float32),
            jax.ShapeDtypeStruct((WIDE, WIDE), jnp.float32),
            jax.ShapeDtypeStruct((1, NUM_TILES), jnp.int32),
        ],
    )(inputs_w, ids)


def _x_body(x_ref, k4_ref, b4_ref, o_ref):
    k4 = k4_ref[...]
    b4 = b4_ref[...]
    for m in range(HALF):
        piece = jnp.dot(x_ref[:, 40 * m:40 * m + 40], k4,
                        preferred_element_type=jnp.float32)
        o_ref[:, m, :] = jnp.maximum(piece + b4, 0.0)


def _x_pass(inputs_w, k4, b4):
    nb = (N // NW) // X_BLOCK
    return pl.pallas_call(
        _x_body,
        grid=(nb,),
        in_specs=[
            pl.BlockSpec((X_BLOCK, WIDE), lambda i: (i, 0)),
            pl.BlockSpec((4 * IN_CH, 128), lambda i: (0, 0)),
            pl.BlockSpec((1, 128), lambda i: (0, 0)),
        ],
        out_specs=pl.BlockSpec((X_BLOCK, HALF, 128), lambda i: (i, 0, 0)),
        out_shape=jax.ShapeDtypeStruct((N // NW, HALF, 128), jnp.float32),
    )(inputs_w, k4, b4)


def _seg_body(x_hbm, ids_hbm, cnt_hbm, out_hbm, table,
              xb0, xb1, ib0, ib1, ob0, ob1, cntbuf,
              sx0, sx1, so0, so1):
    wid = lax.axis_index("s") * NUM_CORES + lax.axis_index("c")
    base = wid * SEGS_PER_TILE

    pltpu.sync_copy(cnt_hbm, cntbuf.at[0:NUM_TILES])
    r0 = jnp.where(
        wid == 0, 0, cntbuf[pl.ds(jnp.maximum(wid - 1, 0), HALF)][0])
    r1 = cntbuf[pl.ds(wid, HALF)][0]
    nrows = r1 - r0
    nch = (nrows + C - 1) // C

    def bases(lo):
        s8 = pl.multiple_of(jnp.minimum((lo // 8) * 8, N - (C + 8)), 8)
        sa = pl.multiple_of(jnp.minimum((lo // 16) * 16, N - IDS_SZ), 16)
        return s8, sa

    def stage_start(lo, xb, ib, sem):
        s8, sa = bases(lo)
        xo = pl.multiple_of(s8 * OUT_CH, 8 * OUT_CH)
        pltpu.make_async_copy(
            x_hbm.at[pl.ds(xo, XW)], xb.at[0:XW], sem).start()
        pltpu.make_async_copy(
            ids_hbm.at[pl.ds(sa, IDS_SZ)], ib.at[0:IDS_SZ], sem).start()

    def stage_wait(lo, xb, ib, sem):
        s8, sa = bases(lo)
        xo = pl.multiple_of(s8 * OUT_CH, 8 * OUT_CH)
        pltpu.make_async_copy(
            x_hbm.at[pl.ds(xo, XW)], xb.at[0:XW], sem).wait()
        pltpu.make_async_copy(
            ids_hbm.at[pl.ds(sa, IDS_SZ)], ib.at[0:IDS_SZ], sem).wait()

    neg = jnp.full((HALF,), -jnp.inf, dtype=jnp.float32)

    def a_lo(k):
        return r0 + k * C

    def a_process(k, xb, ib, carry):
        lo = a_lo(k)
        hi = jnp.minimum(lo + C, r1)
        s8, sa = bases(lo)

        def a_group(j, carry):
            cur, m0, m1 = carry
            g0 = lo + j * HALF
            iv = ib[pl.ds(g0 - sa, HALF)]
            xo0 = (g0 - s8) * OUT_CH
            for t in range(HALF):
                sid = iv[t]
                new = sid != cur
                x0 = xb[pl.ds(xo0 + t * OUT_CH, HALF)]
                x1 = xb[pl.ds(xo0 + t * OUT_CH + HALF, HALF)]
                m0 = jnp.where(new, x0, jnp.maximum(m0, x0))
                m1 = jnp.where(new, x1, jnp.maximum(m1, x1))
                to = (sid - base) * OUT_CH
                table[pl.ds(to, HALF)] = m0
                table[pl.ds(to + HALF, HALF)] = m1
                cur = sid
            return (cur, m0, m1)

        ng = (hi - lo) // HALF
        carry = lax.fori_loop(0, ng, a_group, carry)

        def a_row(g, carry):
            cur, m0, m1 = carry
            sid = ib[pl.ds(g - sa, HALF)][0]
            new = sid != cur
            xo = (g - s8) * OUT_CH
            x0 = xb[pl.ds(xo, HALF)]
            x1 = xb[pl.ds(xo + HALF, HALF)]
            m0 = jnp.where(new, x0, jnp.maximum(m0, x0))
            m1 = jnp.where(new, x1, jnp.maximum(m1, x1))
            to = (sid - base) * OUT_CH
            table[pl.ds(to, HALF)] = m0
            table[pl.ds(to + HALF, HALF)] = m1
            return (sid, m0, m1)

        return lax.fori_loop(lo + ng * HALF, hi, a_row, carry)

    @pl.when(nch > 0)
    def _phase_a():
        stage_start(a_lo(0), xb0, ib0, sx0)

        def a_pair(p, carry):
            k = p * 2

            stage_wait(a_lo(k), xb0, ib0, sx0)

            @pl.when(k + 1 < nch)
            def _():
                stage_start(a_lo(k + 1), xb1, ib1, sx1)

            carry2 = a_process(k, xb0, ib0, carry)

            def odd(c):
                stage_wait(a_lo(k + 1), xb1, ib1, sx1)

                @pl.when(k + 2 < nch)
                def _():
                    stage_start(a_lo(k + 2), xb0, ib0, sx0)

                return a_process(k + 1, xb1, ib1, c)

            return lax.fori_loop(
                0, jnp.where(k + 1 < nch, 1, 0), lambda _, c: odd(c), carry2)

        lax.fori_loop(0, (nch + 1) // 2, a_pair, (jnp.int32(-1), neg, neg))

    nfull = nrows // C
    rem = nrows - nfull * C
    nb_chunks = jnp.where(rem > 0, nfull + 1, nfull)

    def b_lo(k):
        return jnp.minimum(r0 + k * C, r1 - C)

    def b_process(k, xb, ib, ob):
        lo = b_lo(k)
        s8, sa = bases(lo)

        def b_group(j, _):
            g0 = lo + j * HALF
            iv = ib[pl.ds(g0 - sa, HALF)]
            xo0 = (g0 - s8) * OUT_CH
            oo0 = (g0 - lo) * OUT_CH
            sid0 = iv[0]
            uniform = iv[HALF - 1] == sid0

            @pl.when(uniform)
            def _fast():
                to = (sid0 - base) * OUT_CH
                t0 = table[pl.ds(to, HALF)]
                t1 = table[pl.ds(to + HALF, HALF)]
                for t in range(HALF):
                    o0 = (xb[pl.ds(xo0 + t * OUT_CH, HALF)] + t0) * 0.5
                    o1 = (xb[pl.ds(xo0 + t * OUT_CH + HALF, HALF)] + t1) * 0.5
                    ob[pl.ds(oo0 + t * OUT_CH, HALF)] = o0
                    ob[pl.ds(oo0 + t * OUT_CH + HALF, HALF)] = o1

            @pl.when(jnp.logical_not(uniform))
            def _slow():
                for t in range(HALF):
                    sid = iv[t]
                    to = (sid - base) * OUT_CH
                    t0 = table[pl.ds(to, HALF)]
                    t1 = table[pl.ds(to + HALF, HALF)]
                    o0 = (xb[pl.ds(xo0 + t * OUT_CH, HALF)] + t0) * 0.5
                    o1 = (xb[pl.ds(xo0 + t * OUT_CH + HALF, HALF)] + t1) * 0.5
                    ob[pl.ds(oo0 + t * OUT_CH, HALF)] = o0
                    ob[pl.ds(oo0 + t * OUT_CH + HALF, HALF)] = o1
            return 0

        lax.fori_loop(0, C // HALF, b_group, 0)

    def b_write_start(k, ob, sem):
        o0 = pl.multiple_of(b_lo(k) * OUT_CH, OUT_CH)
        pltpu.make_async_copy(
            ob, out_hbm.at[pl.ds(o0, C * OUT_CH)], sem).start()

    def b_write_wait(k, ob, sem):
        o0 = pl.multiple_of(b_lo(k) * OUT_CH, OUT_CH)
        pltpu.make_async_copy(
            ob, out_hbm.at[pl.ds(o0, C * OUT_CH)], sem).wait()

    @pl.when(nrows >= C)
    def _b_main():
        stage_start(b_lo(0), xb0, ib0, sx0)

        def b_pair(p, _):
            k = p * 2

            stage_wait(b_lo(k), xb0, ib0, sx0)

            @pl.when(k + 1 < nb_chunks)
            def _():
                stage_start(b_lo(k + 1), xb1, ib1, sx1)

            @pl.when(k >= 2)
            def _():
                b_write_wait(k - 2, ob0, so0)

            b_process(k, xb0, ib0, ob0)
            b_write_start(k, ob0, so0)

            @pl.when(k + 1 < nb_chunks)
            def _odd():
                stage_wait(b_lo(k + 1), xb1, ib1, sx1)

                @pl.when(k + 2 < nb_chunks)
                def _():
                    stage_start(b_lo(k + 2), xb0, ib0, sx0)

                @pl.when(k >= 1)
                def _():
                    b_write_wait(k - 1, ob1, so1)

                b_process(k + 1, xb1, ib1, ob1)
                b_write_start(k + 1, ob1, so1)

            return 0

        lax.fori_loop(0, (nb_chunks + 1) // 2, b_pair, 0)

        last = nb_chunks - 1

        @pl.when(last % 2 == 0)
        def _():
            b_write_wait(last, ob0, so0)

        @pl.when((last >= 1) & (last % 2 == 1))
        def _():
            b_write_wait(last, ob1, so1)

        @pl.when((last >= 1) & (last % 2 == 0))
        def _():
            b_write_wait(last - 1, ob1, so1)

        @pl.when((last >= 2) & (last % 2 == 1))
        def _():
            b_write_wait(last - 1, ob0, so0)

    @pl.when((nrows > 0) & (nrows < C))
    def _b_small():
        s8, sa = bases(r0)
        stage_start(r0, xb0, ib0, sx0)
        stage_wait(r0, xb0, ib0, sx0)

        def row(g, c):
            sid = ib0[pl.ds(g - sa, HALF)][0]
            to = (sid - base) * OUT_CH
            t0 = table[pl.ds(to, HALF)]
            t1 = table[pl.ds(to + HALF, HALF)]
            xo = (g - s8) * OUT_CH
            oo = pl.multiple_of((g - r0) * OUT_CH, OUT_CH)
            ob0[pl.ds(oo, HALF)] = (xb0[pl.ds(xo, HALF)] + t0) * 0.5
            ob0[pl.ds(oo + HALF, HALF)] = (xb0[pl.ds(xo + HALF, HALF)] + t1) * 0.5
            dst = pl.multiple_of(g * OUT_CH, OUT_CH)
            pltpu.sync_copy(ob0.at[pl.ds(oo, OUT_CH)],
                            out_hbm.at[pl.ds(dst, OUT_CH)])
            return c

        lax.fori_loop(r0, r1, row, 0)


@functools.partial(
    pl.kernel,
    out_type=jax.ShapeDtypeStruct((N * OUT_CH,), jnp.float32),
    mesh=plsc.VectorSubcoreMesh(
        core_axis_name="c", subcore_axis_name="s", num_cores=NUM_CORES,
        num_subcores=NUM_SUBCORES),
    scratch_types=[
        pltpu.VMEM((SEGS_PER_TILE * OUT_CH,), jnp.float32),
        pltpu.VMEM((XW,), jnp.float32),
        pltpu.VMEM((XW,), jnp.float32),
        pltpu.VMEM((IDS_SZ + HALF,), jnp.int32),
        pltpu.VMEM((IDS_SZ + HALF,), jnp.int32),
        pltpu.VMEM((C * OUT_CH,), jnp.float32),
        pltpu.VMEM((C * OUT_CH,), jnp.float32),
        pltpu.VMEM((NUM_TILES + HALF,), jnp.int32),
        pltpu.SemaphoreType.DMA,
        pltpu.SemaphoreType.DMA,
        pltpu.SemaphoreType.DMA,
        pltpu.SemaphoreType.DMA,
    ],
)
def _seg_kernel(x_hbm, ids_hbm, cnt_hbm, out_hbm, table,
                xb0, xb1, ib0, ib1, ob0, ob1, cntbuf, sx0, sx1, so0, so1):
    _seg_body(x_hbm, ids_hbm, cnt_hbm, out_hbm, table,
              xb0, xb1, ib0, ib1, ob0, ob1, cntbuf, sx0, sx1, so0, so1)


def kernel(inputs, unq_inv, W, b, gamma, beta):
    ids = unq_inv.astype(jnp.int32)
    wt = W.T
    inputs_w = inputs.reshape(N // NW, WIDE)

    s640, g640, cnt = _stats_pass(inputs_w, ids)
    sx = s640.reshape(NW, IN_CH).sum(axis=0)
    g10 = g640.reshape(NW, IN_CH, NW, IN_CH)
    g10 = jnp.einsum("jkjl->kl", g10)

    mean = sx @ wt / N + b
    ev2 = jnp.sum((g10 @ wt) * wt, axis=0) / N
    var = ev2 - (sx @ wt / N) ** 2
    a = gamma * lax.rsqrt(var + EPS)
    d = beta - mean * a
    wt2 = wt * a[None, :]
    b2 = b * a + d

    k4 = jnp.zeros((4 * IN_CH, 128), jnp.float32)
    for u in range(4):
        k4 = k4.at[IN_CH * u:IN_CH * (u + 1),
                   OUT_CH * u:OUT_CH * (u + 1)].set(wt2)
    b4 = jnp.tile(b2, 4).reshape(1, 128)

    x = _x_pass(inputs_w, k4, b4)

    out_flat = _seg_kernel(x.reshape(N * OUT_CH), ids, cnt[0])
    return out_flat.reshape(N, OUT_CH)
